# baseline (speedup 1.0000x reference)
import functools
import jax
import jax.numpy as jnp
from jax import lax
from jax.experimental import pallas as pl
from jax.experimental.pallas import tpu as pltpu, tpu_sc as plsc

_NC = 2
_NS = 16
_NW = _NC * _NS
_LANES = 16


def _policy_lr_sc(rows, cols, L, Rflat, log_sigma, M):
    B = rows.shape[0]
    K = L.shape[1]
    assert B % _NW == 0 and K == 2 * _LANES
    NB = B // _NW
    NCH = (NB * K) // 128
    BPC = 128 // K

    mesh = plsc.VectorSubcoreMesh(
        core_axis_name="c", subcore_axis_name="s",
        num_cores=_NC, num_subcores=_NS)

    @functools.partial(
        pl.kernel,
        out_type=(jax.ShapeDtypeStruct((B,), jnp.float32),
                  jax.ShapeDtypeStruct((_LANES,), jnp.float32)),
        mesh=mesh,
        compiler_params=pltpu.CompilerParams(
            needs_layout_passes=False, use_tc_tiling_on_sc=False),
        scratch_types=[
            pltpu.VMEM((BPC, 128), jnp.int32),
            pltpu.VMEM((NB + _LANES,), jnp.int32),
            pltpu.VMEM((NB, K), jnp.float32),
            pltpu.VMEM((NCH, 128), jnp.int32),
            pltpu.VMEM((NCH, 128), jnp.float32),
            pltpu.VMEM((NB,), jnp.float32),
            pltpu.VMEM((_LANES,), jnp.float32),
            pltpu.SemaphoreType.DMA,
            pltpu.SemaphoreType.DMA,
        ],
    )
    def k(rows_h, cols_h, l_h, rf_h, sig_h, out_h, out2_h,
          rows_v, cols_v, l_v, idx_v, r_v, res_v, sig_v, lsem, rsem):
        wid = lax.axis_index("s") * _NC + lax.axis_index("c")
        base = wid * NB

        for i in range(BPC):
            pltpu.sync_copy(rows_h.at[pl.ds(base + i * 128, 128)],
                            rows_v.at[i])
        pltpu.sync_copy(cols_h.at[pl.ds(base, NB)], cols_v.at[pl.ds(0, NB)])

        l_copies = [
            pltpu.async_copy(l_h.at[rows_v.at[i]],
                             l_v.at[pl.ds(i * 128, 128)], lsem)
            for i in range(BPC)
        ]

        km0 = lax.iota(jnp.int32, _LANES) * M
        km1 = km0 + _LANES * M

        def build(c, _):
            for q in range(BPC):
                bi = c * BPC + q
                cv = cols_v[pl.ds(bi, _LANES)]
                vc = jnp.full((_LANES,), cv[0], jnp.int32)
                idx_v[c, pl.ds(q * K, _LANES)] = vc + km0
                idx_v[c, pl.ds(q * K + _LANES, _LANES)] = vc + km1
            return 0

        lax.fori_loop(0, NCH, build, 0)

        r_copies = [
            pltpu.async_copy(rf_h.at[idx_v.at[c]], r_v.at[c], rsem)
            for c in range(NCH)
        ]

        for cp in l_copies:
            cp.wait()
        for cp in r_copies:
            cp.wait()

        lane = lax.iota(jnp.int32, _LANES)
        last = lane == (_LANES - 1)

        def comp(c, _):
            for q in range(BPC):
                bi = c * BPC + q
                l0 = l_v[bi, pl.ds(0, _LANES)]
                l1 = l_v[bi, pl.ds(_LANES, _LANES)]
                r0 = r_v[c, pl.ds(q * K, _LANES)]
                r1 = r_v[c, pl.ds(q * K + _LANES, _LANES)]
                v = l0 * r0 + l1 * r1
                cs = plsc.cumsum(v)
                plsc.store_scatter(
                    res_v, [jnp.full((_LANES,), bi, jnp.int32)], cs,
                    mask=last)
            return 0

        lax.fori_loop(0, NCH, comp, 0)

        pltpu.sync_copy(res_v, out_h.at[pl.ds(base, NB)])

        @pl.when(wid == 0)
        def _():
            pltpu.sync_copy(sig_h, sig_v.at[pl.ds(0, 1)])
            v = sig_v[...]
            sig_v[...] = jnp.minimum(jnp.maximum(v, -2.5), 0.0)
            pltpu.sync_copy(sig_v, out2_h)

    return k(rows, cols, L, Rflat, log_sigma)


def kernel(rows, cols, L, R, log_sigma):
    M = R.shape[1]
    res, sig16 = _policy_lr_sc(rows.astype(jnp.int32), cols.astype(jnp.int32),
                               L, R.reshape(-1), log_sigma, M)
    return res, sig16[:1]
